# baseline (speedup 1.0000x reference)
import functools

import jax
import jax.numpy as jnp
from jax.experimental import pallas as pl
from jax.experimental.pallas import tpu as pltpu
from jax.experimental.pallas import tpu_sc as plsc

_NC, _NS, _L = 2, 16, 16
_NW = _NC * _NS


def _gelu(x):
    return 0.5 * x * (1.0 + jax.lax.erf(x * (2.0 ** -0.5)))


def _gather_heads(seq, bpw, sem_hbm, out_hbm, rows_v, dma_sem):
    wid = jax.lax.axis_index("s") * _NC + jax.lax.axis_index("c")
    base = wid * bpw
    copies = []
    for j in range(bpw):
        cp = pltpu.make_async_copy(
            sem_hbm.at[base + j, pl.ds(0, 1), :],
            rows_v.at[pl.ds(j, 1), :], dma_sem)
        cp.start()
        copies.append(cp)
    for cp in copies:
        cp.wait()
    pltpu.sync_copy(rows_v, out_hbm.at[pl.ds(base, bpw)])


def _xattn_kernel(adj_ref, sem0_ref, w_ref, w1_ref, g_ref, b_ref, w2_ref,
                  out_ref):
    h = jnp.dot(sem0_ref[:], w_ref[:], preferred_element_type=jnp.float32)
    ones = jnp.ones((h.shape[0], 1), jnp.float32)
    hx = jnp.concatenate([h, ones], axis=1)
    mask = (adj_ref[:] != 0).astype(jnp.float32)
    agg = jax.lax.dot_general(
        mask, hx, (((0,), (0,)), ((), ())),
        preferred_element_type=jnp.float32)
    d = h.shape[1]
    sums = agg[:, :d]
    counts = agg[:, d:d + 1]
    x = _gelu(sums / jnp.maximum(counts, 1.0))
    x = jax.lax.dot_general(
        x, w1_ref[:], (((1,), (1,)), ((), ())),
        preferred_element_type=jnp.float32)
    x = _gelu(x)
    mu = jnp.mean(x, axis=-1, keepdims=True)
    var = jnp.mean((x - mu) ** 2, axis=-1, keepdims=True)
    x = (x - mu) / jnp.sqrt(var + 1e-5) * g_ref[:] + b_ref[:]
    out_ref[:] = jax.lax.dot_general(
        x, w2_ref[:], (((1,), (1,)), ((), ())),
        preferred_element_type=jnp.float32)


@jax.jit
def kernel(adj, semantics, attention_masks, W, W1, ln_g, ln_b, W2):
    del attention_masks
    n, seq, d = semantics.shape
    bpw = n // _NW
    mesh = plsc.VectorSubcoreMesh(
        core_axis_name="c", subcore_axis_name="s",
        num_cores=_NC, num_subcores=_NS)
    sem0 = pl.kernel(
        functools.partial(_gather_heads, seq, bpw),
        out_type=jax.ShapeDtypeStruct((n, d), jnp.float32),
        mesh=mesh,
        scratch_types=[
            pltpu.VMEM((bpw, d), jnp.float32),
            pltpu.SemaphoreType.DMA,
        ],
    )(semantics)
    out = pl.pallas_call(
        _xattn_kernel,
        out_shape=jax.ShapeDtypeStruct((n, 1), jnp.float32),
    )(adj, sem0, W, W1, ln_g.reshape(1, d), ln_b.reshape(1, d), W2)
    return out[:, 0]


# speedup vs baseline: 2.5940x; 2.5940x over previous
import jax
import jax.numpy as jnp
from jax.experimental import pallas as pl


def _gelu(x):
    return 0.5 * x * (1.0 + jax.lax.erf(x * (2.0 ** -0.5)))


def _xattn_kernel(adj_ref, sem8_ref, w_ref, w1_ref, g_ref, b_ref, w2_ref,
                  out_ref):
    n = adj_ref.shape[0]
    d = w_ref.shape[0]
    sem0 = sem8_ref[:, 0, :].reshape(n, d)
    h = jnp.dot(sem0, w_ref[:], preferred_element_type=jnp.float32)
    ones = jnp.ones((n, 1), jnp.float32)
    hx = jnp.concatenate([h, ones], axis=1)
    mask = (adj_ref[:] != 0).astype(jnp.float32)
    agg = jax.lax.dot_general(
        mask, hx, (((0,), (0,)), ((), ())),
        preferred_element_type=jnp.float32)
    sums = agg[:, :d]
    counts = agg[:, d:d + 1]
    x = _gelu(sums / jnp.maximum(counts, 1.0))
    x = jax.lax.dot_general(
        x, w1_ref[:], (((1,), (1,)), ((), ())),
        preferred_element_type=jnp.float32)
    x = _gelu(x)
    mu = jnp.mean(x, axis=-1, keepdims=True)
    var = jnp.mean((x - mu) ** 2, axis=-1, keepdims=True)
    x = (x - mu) / jnp.sqrt(var + 1e-5) * g_ref[:] + b_ref[:]
    out_ref[:] = jax.lax.dot_general(
        x, w2_ref[:], (((1,), (1,)), ((), ())),
        preferred_element_type=jnp.float32)


@jax.jit
def kernel(adj, semantics, attention_masks, W, W1, ln_g, ln_b, W2):
    del attention_masks
    n, seq, d = semantics.shape
    sem8 = jax.lax.slice(semantics, (0, 0, 0), (n, 8, d))
    out = pl.pallas_call(
        _xattn_kernel,
        out_shape=jax.ShapeDtypeStruct((n, 1), jnp.float32),
    )(adj, sem8, W, W1, ln_g.reshape(1, d), ln_b.reshape(1, d), W2)
    return out[:, 0]


# speedup vs baseline: 2.7907x; 1.0758x over previous
import jax
import jax.numpy as jnp
from jax.experimental import pallas as pl


def _gelu(x):
    return 0.5 * x * (1.0 + jax.lax.erf(x * (2.0 ** -0.5)))


def _xattn_kernel(adj_ref, sem0_ref, w_ref, w1_ref, g_ref, b_ref, w2_ref,
                  out_ref):
    h = jnp.dot(sem0_ref[:], w_ref[:], preferred_element_type=jnp.float32)
    ones = jnp.ones((h.shape[0], 1), jnp.float32)
    hx = jnp.concatenate([h, ones], axis=1)
    mask = (adj_ref[:] != 0).astype(jnp.float32)
    agg = jax.lax.dot_general(
        mask, hx, (((0,), (0,)), ((), ())),
        preferred_element_type=jnp.float32)
    d = h.shape[1]
    sums = agg[:, :d]
    counts = agg[:, d:d + 1]
    x = _gelu(sums / jnp.maximum(counts, 1.0))
    x = jax.lax.dot_general(
        x, w1_ref[:], (((1,), (1,)), ((), ())),
        preferred_element_type=jnp.float32)
    x = _gelu(x)
    mu = jnp.mean(x, axis=-1, keepdims=True)
    var = jnp.mean((x - mu) ** 2, axis=-1, keepdims=True)
    x = (x - mu) / jnp.sqrt(var + 1e-5) * g_ref[:] + b_ref[:]
    out_ref[:] = jax.lax.dot_general(
        x, w2_ref[:], (((1,), (1,)), ((), ())),
        preferred_element_type=jnp.float32)


@jax.jit
def kernel(adj, semantics, attention_masks, W, W1, ln_g, ln_b, W2):
    del attention_masks
    n = adj.shape[0]
    d = W.shape[0]
    sem0 = semantics[:, 0, :]
    out = pl.pallas_call(
        _xattn_kernel,
        out_shape=jax.ShapeDtypeStruct((n, 1), jnp.float32),
    )(adj, sem0, W, W1, ln_g.reshape(1, d), ln_b.reshape(1, d), W2)
    return out[:, 0]
